# baseline (speedup 1.0000x reference)
import math
import functools

import jax
import jax.numpy as jnp
from jax.experimental import pallas as pl
from jax.experimental.pallas import tpu as pltpu

N_EDGES = 160000
E_EDGE = 256
E_QIN = 32
E_RBF = 16
BLK_E = 1000


def _stage1_body(m_ref, br_ref, wdb_ref, wrbf_ref, wdown_ref, out_ref):
    t = jnp.dot(m_ref[...], wdb_ref[...], preferred_element_type=jnp.float32)
    rb = jnp.dot(br_ref[...], wrbf_ref[...], preferred_element_type=jnp.float32)
    out_ref[...] = jnp.dot(t * rb, wdown_ref[...], preferred_element_type=jnp.float32)


def _stage1(m, bases_rad, W_db, W_rbf, W_down):
    n = m.shape[0]
    grid = (n // BLK_E,)
    return pl.pallas_call(
        _stage1_body,
        grid=grid,
        in_specs=[
            pl.BlockSpec((BLK_E, E_EDGE), lambda i: (i, 0)),
            pl.BlockSpec((BLK_E, E_RBF), lambda i: (i, 0)),
            pl.BlockSpec((E_EDGE, E_EDGE), lambda i: (0, 0)),
            pl.BlockSpec((E_RBF, E_EDGE), lambda i: (0, 0)),
            pl.BlockSpec((E_EDGE, E_QIN), lambda i: (0, 0)),
        ],
        out_specs=pl.BlockSpec((BLK_E, E_QIN), lambda i: (i, 0)),
        out_shape=jax.ShapeDtypeStruct((n, E_QIN), jnp.float32),
    )(m, bases_rad, W_db, W_rbf, W_down)


def kernel(m, bases_rad, bases_cir, sph_rbf_W1, sph_sph, idx_triplet_in_in,
           idx_trip_in_to_quad, idx_out, idx_out_agg, id_swap,
           W_db, W_rbf, W_cbf, W_down, W_bil, W_up_ca, W_up_ac):
    inv_sqrt_2 = 1.0 / math.sqrt(2.0)
    x_db = _stage1(m, bases_rad, W_db, W_rbf, W_down)
    x_db = jnp.take(x_db, idx_triplet_in_in, axis=0)
    x_db = x_db * (bases_cir @ W_cbf)
    x_db = jnp.take(x_db, idx_trip_in_to_quad, axis=0)
    nE = sph_rbf_W1.shape[0]
    Kmax = sph_sph.shape[1]
    m_pad = jnp.zeros((nE, Kmax, x_db.shape[-1]), x_db.dtype).at[idx_out, idx_out_agg].set(x_db)
    sph_m = jnp.matmul(jnp.swapaxes(sph_sph, -1, -2), m_pad)
    r = jnp.matmul(sph_rbf_W1, sph_m)
    x = r.reshape(nE, -1) @ W_bil
    x_ca = x @ W_up_ca
    x_ac = jnp.take(x @ W_up_ac, id_swap, axis=0)
    return (x_ca + x_ac) * inv_sqrt_2
